# baseline (speedup 1.0000x reference)
import functools

import jax
import jax.numpy as jnp
from jax import lax
from jax.experimental import pallas as pl
from jax.experimental.pallas import tpu as pltpu
from jax.experimental.pallas import tpu_sc as plsc

_THRES = 0.7
_MIN_KEPT = 131072
_N = 1048576
_LANES = 16
_NS = 16
_CHUNK = _N // _NS
_UNROLL = 32
_NACC = 8
_NBUF = 4
_BUF = _CHUNK // _NBUF


def _sigmoid(x):
    return 1.0 / (1.0 + jnp.exp(-x))


def _log1p_exp_neg(a):
    u = jnp.exp(-a)
    y = u
    for _ in range(5):
        y = y - 1.0 + (1.0 + u) * jnp.exp(-y)
    return y


def _sc_body(score_hbm, tgt_hbm, parts_hbm, out_hbm, tgt_v, sc8_v, part_v,
             all_v, res_v, t2_v, sems):
    sid = lax.axis_index("s")
    base = sid * _CHUNK

    copies = [
        pltpu.async_copy(
            tgt_hbm.at[pl.ds(base + b * _BUF, _BUF)],
            tgt_v.at[pl.ds(b * _BUF, _BUF)], sems.at[b])
        for b in range(2)
    ]

    z = jnp.zeros((_LANES,), jnp.int32)

    def reduce_buf(buf_base, carry):
        def step(i, carry):
            accs = list(carry)
            off = buf_base + i * (_LANES * _UNROLL)
            for k in range(_UNROLL):
                accs[k % _NACC] = accs[k % _NACC] \
                    + tgt_v[pl.ds(off + k * _LANES, _LANES)]
            return tuple(accs)
        return lax.fori_loop(0, _BUF // (_LANES * _UNROLL), step, carry)

    carry = (z,) * _NACC
    for j in range(_NBUF):
        copies[j % 2].wait()
        carry = reduce_buf((j % 2) * _BUF, carry)
        if j + 2 < _NBUF:
            copies[j % 2] = pltpu.async_copy(
                tgt_hbm.at[pl.ds(base + (j + 2) * _BUF, _BUF)],
                tgt_v.at[pl.ds((j % 2) * _BUF, _BUF)], sems.at[j % 2])
    acc = carry[0]
    for a in carry[1:]:
        acc = acc + a
    part_v[...] = acc

    pltpu.sync_copy(part_v, parts_hbm.at[sid])
    plsc.subcore_barrier()

    @pl.when(sid == 0)
    def _():
        pltpu.sync_copy(parts_hbm, all_v)
        tot = all_v[0]
        for i in range(1, _NS):
            tot = tot + all_v[i]
        part_v[...] = tot
        n1v = plsc.load_gather(part_v, [jnp.zeros((_LANES,), jnp.int32)])
        for k in range(1, _LANES):
            n1v = n1v + plsc.load_gather(
                part_v, [jnp.full((_LANES,), k, jnp.int32)])
        n0v = jnp.full((_LANES,), _N, jnp.int32) - n1v

        pltpu.sync_copy(score_hbm, sc8_v)
        zi = jnp.zeros((_LANES,), jnp.int32)
        oi = jnp.full((_LANES,), 1, jnp.int32)
        s00 = plsc.load_gather(sc8_v, [zi])
        s01 = plsc.load_gather(sc8_v, [oi])
        s10 = plsc.load_gather(sc8_v, [jnp.full((_LANES,), 2, jnp.int32)])
        s11 = plsc.load_gather(sc8_v, [jnp.full((_LANES,), 3, jnp.int32)])
        pltpu.sync_copy(tgt_hbm.at[pl.ds(0, _LANES)], t2_v)
        t0 = plsc.load_gather(t2_v, [zi])
        t1 = plsc.load_gather(t2_v, [oi])
        t0f = t0.astype(jnp.float32)

        A = _sigmoid(jnp.where(t0 == 0, s00, s01))
        B = _sigmoid(jnp.where(t1 == 0, s10, s11))
        P0 = jnp.maximum(s00, 0.0) - s00 * t0f + _log1p_exp_neg(jnp.abs(s00))
        P1 = jnp.maximum(s01, 0.0) - s01 * t0f + _log1p_exp_neg(jnp.abs(s01))

        kq = jnp.full((_LANES,), _MIN_KEPT, jnp.int32)
        min_value = jnp.where(
            A < B,
            jnp.where(n0v > kq, A, B),
            jnp.where(A > B, jnp.where(n1v > kq, B, A), A),
        )
        thr = jnp.maximum(min_value, _THRES)
        zf = jnp.zeros((_LANES,), jnp.float32)
        n0f = n0v.astype(jnp.float32)
        n1f = n1v.astype(jnp.float32)
        kA = jnp.where(A < thr, n0f, zf)
        kB = jnp.where(B < thr, n1f, zf)
        res = (P0 * kA + P1 * kB) / jnp.maximum(kA + kB, 1.0)
        res_v[...] = res
        pltpu.sync_copy(res_v, out_hbm)


@jax.jit
def kernel(score, target):
    score = jnp.reshape(lax.slice(score, (0, 0), (8, 2)), (_LANES,))
    mesh = plsc.VectorSubcoreMesh(
        core_axis_name="c", subcore_axis_name="s", num_cores=1)
    _, out = pl.kernel(
        _sc_body,
        out_type=(
            jax.ShapeDtypeStruct((_NS, _LANES), jnp.int32),
            jax.ShapeDtypeStruct((_LANES,), jnp.float32),
        ),
        mesh=mesh,
        compiler_params=pltpu.CompilerParams(needs_layout_passes=False),
        scratch_types=[
            pltpu.VMEM((2 * _BUF,), jnp.int32),
            pltpu.VMEM((_LANES,), jnp.float32),
            pltpu.VMEM((_LANES,), jnp.int32),
            pltpu.VMEM((_NS, _LANES), jnp.int32),
            pltpu.VMEM((_LANES,), jnp.float32),
            pltpu.VMEM((_LANES,), jnp.int32),
            pltpu.SemaphoreType.DMA((2,)),
        ],
    )(score, target)
    return out[0]


# speedup vs baseline: 1.0050x; 1.0050x over previous
import jax
import jax.numpy as jnp
from jax import lax
from jax.experimental import pallas as pl
from jax.experimental.pallas import tpu as pltpu
from jax.experimental.pallas import tpu_sc as plsc

_THRES = 0.7
_MIN_KEPT = 131072
_N = 1048576
_LANES = 16
_NC = 2
_NS = 16
_NW = _NC * _NS
_CHUNK = _N // _NW
_UNROLL = 8
_NBUF = 4
_BUF = _CHUNK // _NBUF


def _sc_body(tgt_hbm, parts_hbm, tgt_v, part_v, sems):
    wid = lax.axis_index("s") * _NC + lax.axis_index("c")
    base = wid * _CHUNK

    copies = [
        pltpu.async_copy(
            tgt_hbm.at[pl.ds(base + b * _BUF, _BUF)],
            tgt_v.at[pl.ds(b * _BUF, _BUF)], sems.at[b])
        for b in range(2)
    ]

    z = jnp.zeros((_LANES,), jnp.int32)

    def reduce_buf(buf_base, carry):
        def step(i, carry):
            a0, a1, a2, a3 = carry
            off = buf_base + i * (_LANES * _UNROLL)
            a0 = a0 + tgt_v[pl.ds(off + 0 * _LANES, _LANES)] \
                    + tgt_v[pl.ds(off + 4 * _LANES, _LANES)]
            a1 = a1 + tgt_v[pl.ds(off + 1 * _LANES, _LANES)] \
                    + tgt_v[pl.ds(off + 5 * _LANES, _LANES)]
            a2 = a2 + tgt_v[pl.ds(off + 2 * _LANES, _LANES)] \
                    + tgt_v[pl.ds(off + 6 * _LANES, _LANES)]
            a3 = a3 + tgt_v[pl.ds(off + 3 * _LANES, _LANES)] \
                    + tgt_v[pl.ds(off + 7 * _LANES, _LANES)]
            return a0, a1, a2, a3
        return lax.fori_loop(0, _BUF // (_LANES * _UNROLL), step, carry)

    carry = (z, z, z, z)
    for j in range(_NBUF):
        copies[j % 2].wait()
        carry = reduce_buf((j % 2) * _BUF, carry)
        if j + 2 < _NBUF:
            copies[j % 2] = pltpu.async_copy(
                tgt_hbm.at[pl.ds(base + (j + 2) * _BUF, _BUF)],
                tgt_v.at[pl.ds((j % 2) * _BUF, _BUF)], sems.at[j % 2])
    a0, a1, a2, a3 = carry
    part_v[...] = (a0 + a1) + (a2 + a3)
    pltpu.sync_copy(part_v, parts_hbm.at[wid])


def _tc_body(parts_ref, score_ref, tgt_ref, out_ref):
    n1 = jnp.sum(parts_ref[...])
    n0 = _N - n1
    t0 = tgt_ref[0]
    t1 = tgt_ref[1]
    s00 = score_ref[0, 0]
    s01 = score_ref[0, 1]
    s10 = score_ref[1, 0]
    s11 = score_ref[1, 1]

    shp = (8, 128)
    t0f = jnp.full(shp, t0.astype(jnp.float32))
    xa = jnp.full(shp, jnp.where(t0 == 0, s00, s01))
    xb = jnp.full(shp, jnp.where(t1 == 0, s10, s11))
    a = jax.nn.sigmoid(xa)
    b = jax.nn.sigmoid(xb)
    v00 = jnp.full(shp, s00)
    v01 = jnp.full(shp, s01)
    p0 = jnp.maximum(v00, 0.0) - v00 * t0f + jnp.log1p(jnp.exp(-jnp.abs(v00)))
    p1 = jnp.maximum(v01, 0.0) - v01 * t0f + jnp.log1p(jnp.exp(-jnp.abs(v01)))

    n0v = jnp.full(shp, n0)
    n1v = jnp.full(shp, n1)
    kq = jnp.full(shp, _MIN_KEPT)
    min_value = jnp.where(
        a < b,
        jnp.where(n0v > kq, a, b),
        jnp.where(a > b, jnp.where(n1v > kq, b, a), a),
    )
    thr = jnp.maximum(min_value, _THRES)
    zf = jnp.zeros(shp, jnp.float32)
    n0f = n0v.astype(jnp.float32)
    n1f = n1v.astype(jnp.float32)
    ka = jnp.where(a < thr, n0f, zf)
    kb = jnp.where(b < thr, n1f, zf)
    out_ref[...] = (p0 * ka + p1 * kb) / jnp.maximum(ka + kb, 1.0)


@jax.jit
def kernel(score, target):
    mesh = plsc.VectorSubcoreMesh(
        core_axis_name="c", subcore_axis_name="s", num_cores=_NC)
    parts = pl.kernel(
        _sc_body,
        out_type=jax.ShapeDtypeStruct((_NW, _LANES), jnp.int32),
        mesh=mesh,
        compiler_params=pltpu.CompilerParams(needs_layout_passes=False),
        scratch_types=[
            pltpu.VMEM((2 * _BUF,), jnp.int32),
            pltpu.VMEM((_LANES,), jnp.int32),
            pltpu.SemaphoreType.DMA((2,)),
        ],
    )(target)

    score8 = lax.slice(score, (0, 0), (8, 2))
    tgt8 = lax.slice(target, (0,), (8,))
    out = pl.pallas_call(
        _tc_body,
        out_shape=jax.ShapeDtypeStruct((8, 128), jnp.float32),
        in_specs=[
            pl.BlockSpec(memory_space=pltpu.VMEM),
            pl.BlockSpec(memory_space=pltpu.SMEM),
            pl.BlockSpec(memory_space=pltpu.SMEM),
        ],
        out_specs=pl.BlockSpec(memory_space=pltpu.VMEM),
    )(parts, score8, tgt8)
    return out[0, 0]


# speedup vs baseline: 1.0478x; 1.0427x over previous
import jax
import jax.numpy as jnp
from jax import lax
from jax.experimental import pallas as pl
from jax.experimental.pallas import tpu as pltpu
from jax.experimental.pallas import tpu_sc as plsc

_THRES = 0.7
_MIN_KEPT = 131072
_N = 1048576
_LANES = 16
_NC = 2
_NS = 16
_NW = _NC * _NS
_CHUNK = _N // _NW
_UNROLL = 8
_NBUF = 4
_BUF = _CHUNK // _NBUF


def _sc_body(tgt_hbm, parts_hbm, tgt_v, part_v, sems):
    wid = lax.axis_index("s") * _NC + lax.axis_index("c")
    base = wid * _CHUNK

    copies = [
        pltpu.async_copy(
            tgt_hbm.at[pl.ds(base + b * _BUF, _BUF)],
            tgt_v.at[pl.ds(b * _BUF, _BUF)], sems.at[b])
        for b in range(2)
    ]

    z = jnp.zeros((_LANES,), jnp.int32)

    def reduce_buf(buf_base, carry):
        def step(i, carry):
            a0, a1, a2, a3 = carry
            off = buf_base + i * (_LANES * _UNROLL)
            a0 = a0 + tgt_v[pl.ds(off + 0 * _LANES, _LANES)] \
                    + tgt_v[pl.ds(off + 4 * _LANES, _LANES)]
            a1 = a1 + tgt_v[pl.ds(off + 1 * _LANES, _LANES)] \
                    + tgt_v[pl.ds(off + 5 * _LANES, _LANES)]
            a2 = a2 + tgt_v[pl.ds(off + 2 * _LANES, _LANES)] \
                    + tgt_v[pl.ds(off + 6 * _LANES, _LANES)]
            a3 = a3 + tgt_v[pl.ds(off + 3 * _LANES, _LANES)] \
                    + tgt_v[pl.ds(off + 7 * _LANES, _LANES)]
            return a0, a1, a2, a3
        return lax.fori_loop(0, _BUF // (_LANES * _UNROLL), step, carry)

    carry = (z, z, z, z)
    for j in range(_NBUF):
        copies[j % 2].wait()
        carry = reduce_buf((j % 2) * _BUF, carry)
        if j + 2 < _NBUF:
            copies[j % 2] = pltpu.async_copy(
                tgt_hbm.at[pl.ds(base + (j + 2) * _BUF, _BUF)],
                tgt_v.at[pl.ds((j % 2) * _BUF, _BUF)], sems.at[j % 2])
    a0, a1, a2, a3 = carry
    part_v[...] = (a0 + a1) + (a2 + a3)
    pltpu.sync_copy(part_v, parts_hbm.at[wid])


def _tc_body(parts_ref, score_ref, tgt_ref, out_ref):
    n1 = jnp.sum(parts_ref[...])
    n0 = _N - n1
    t0 = tgt_ref[0]
    t1 = tgt_ref[1]
    s00 = score_ref[0, 0]
    s01 = score_ref[0, 1]
    s10 = score_ref[1, 0]
    s11 = score_ref[1, 1]

    t0f = t0.astype(jnp.float32)
    a = jax.nn.sigmoid(jnp.where(t0 == 0, s00, s01))
    b = jax.nn.sigmoid(jnp.where(t1 == 0, s10, s11))
    p0 = jnp.maximum(s00, 0.0) - s00 * t0f + jnp.log1p(jnp.exp(-jnp.abs(s00)))
    p1 = jnp.maximum(s01, 0.0) - s01 * t0f + jnp.log1p(jnp.exp(-jnp.abs(s01)))

    min_value = jnp.where(
        a < b,
        jnp.where(n0 > _MIN_KEPT, a, b),
        jnp.where(a > b, jnp.where(n1 > _MIN_KEPT, b, a), a),
    )
    thr = jnp.maximum(min_value, _THRES)
    n0f = n0.astype(jnp.float32)
    n1f = n1.astype(jnp.float32)
    ka = jnp.where(a < thr, n0f, 0.0)
    kb = jnp.where(b < thr, n1f, 0.0)
    out_ref[0] = (p0 * ka + p1 * kb) / jnp.maximum(ka + kb, 1.0)


@jax.jit
def kernel(score, target):
    mesh = plsc.VectorSubcoreMesh(
        core_axis_name="c", subcore_axis_name="s", num_cores=_NC)
    parts = pl.kernel(
        _sc_body,
        out_type=jax.ShapeDtypeStruct((_NW, _LANES), jnp.int32),
        mesh=mesh,
        compiler_params=pltpu.CompilerParams(needs_layout_passes=False),
        scratch_types=[
            pltpu.VMEM((2 * _BUF,), jnp.int32),
            pltpu.VMEM((_LANES,), jnp.int32),
            pltpu.SemaphoreType.DMA((2,)),
        ],
    )(target)

    score8 = lax.slice(score, (0, 0), (8, 2))
    tgt8 = lax.slice(target, (0,), (8,))
    out = pl.pallas_call(
        _tc_body,
        out_shape=jax.ShapeDtypeStruct((1,), jnp.float32),
        in_specs=[
            pl.BlockSpec(memory_space=pltpu.VMEM),
            pl.BlockSpec(memory_space=pltpu.SMEM),
            pl.BlockSpec(memory_space=pltpu.SMEM),
        ],
        out_specs=pl.BlockSpec(memory_space=pltpu.SMEM),
    )(parts, score8, tgt8)
    return jnp.reshape(out, ())
